# baseline (speedup 1.0000x reference)
import functools

import jax
import jax.numpy as jnp
from jax import lax
from jax.experimental import pallas as pl
from jax.experimental.pallas import tpu as pltpu
from jax.experimental.pallas import tpu_sc as plsc

N_NODES = 10000
N_EDGES = 160000
D_NODE = 128
D_EDGE = 16
H = 32
EH = 64
OUT = 12

NC = 2
NS = 16
NW = NC * NS
CHUNK = 128
NCH = 40
HALF = NCH // 2
EPW = NCH * CHUNK
EP = NW * EPW
EB = 1024
PZ = 10016
RPS = PZ // NS


def _mesh():
    return plsc.VectorSubcoreMesh(
        core_axis_name="c", subcore_axis_name="s", num_cores=NC, num_subcores=NS)


@functools.cache
def _sc_gather():
    @functools.partial(
        pl.kernel,
        out_type=jax.ShapeDtypeStruct((EP, H), jnp.float32),
        mesh=_mesh(),
        compiler_params=pltpu.CompilerParams(use_tc_tiling_on_sc=False),
        scratch_types=[
            pltpu.VMEM((EPW,), jnp.int32),
            pltpu.VMEM((HALF * CHUNK, H), jnp.float32),
            pltpu.SemaphoreType.DMA,
        ],
    )
    def gather(tbl_hbm, idx_hbm, out_hbm, idx_v, rows_v, sem):
        c = lax.axis_index("c")
        s = lax.axis_index("s")
        w = s * NC + c
        pltpu.sync_copy(idx_hbm.at[pl.ds(w * EPW, EPW)], idx_v)
        for half in range(2):
            hc = HALF * CHUNK
            cp = pltpu.async_copy(
                tbl_hbm.at[idx_v.at[pl.ds(half * hc, hc)]], rows_v, sem)
            cp.wait()
            pltpu.sync_copy(
                rows_v, out_hbm.at[pl.ds(w * EPW + half * hc, hc)])

    return gather


@functools.cache
def _sc_scatter():
    @functools.partial(
        pl.kernel,
        out_type=jax.ShapeDtypeStruct((NC, PZ, H), jnp.float32),
        mesh=_mesh(),
        compiler_params=pltpu.CompilerParams(use_tc_tiling_on_sc=False),
        scratch_types=[
            pltpu.VMEM((NCH, CHUNK), jnp.int32),
            pltpu.VMEM((HALF * CHUNK, H), jnp.float32),
            pltpu.VMEM_SHARED((PZ, H), jnp.float32),
            pltpu.SemaphoreType.DMA,
        ],
    )
    def scatter(msg_hbm, idx_hbm, zeros_hbm, out_hbm, idx_v, msg_v, acc_sh, sem):
        c = lax.axis_index("c")
        s = lax.axis_index("s")
        w = s * NC + c
        pltpu.sync_copy(zeros_hbm.at[pl.ds(s * RPS, RPS)],
                        acc_sh.at[pl.ds(s * RPS, RPS)])
        pltpu.sync_copy(idx_hbm.at[w], idx_v)
        plsc.subcore_barrier()
        for half in range(2):
            hc = HALF * CHUNK
            cp = pltpu.async_copy(
                msg_hbm.at[pl.ds(w * EPW + half * hc, hc)], msg_v, sem)
            cp.wait()
            cps = []
            for b in range(HALF):
                cps.append(pltpu.async_copy(
                    msg_v.at[pl.ds(b * CHUNK, CHUNK)],
                    acc_sh.at[idx_v.at[half * HALF + b]], sem,
                    add=True))
            for cp2 in cps:
                cp2.wait()
        plsc.subcore_barrier()
        pltpu.sync_copy(acc_sh.at[pl.ds(s * RPS, RPS)],
                        out_hbm.at[c, pl.ds(s * RPS, RPS)])

    return scatter


def _init_body(x_ref, w_ref, b_ref, o_ref):
    o_ref[...] = jnp.maximum(
        jnp.dot(x_ref[...], w_ref[...], preferred_element_type=jnp.float32)
        + b_ref[...], 0.0)


@functools.cache
def _tc_init():
    return pl.pallas_call(
        _init_body,
        out_shape=jax.ShapeDtypeStruct((N_NODES, H), jnp.float32))


def _msg_body(ef_ref, hs_ref, we1t_ref, be1_ref, we2t_ref, rm_ref, bb_ref,
              msg_ref):
    g = jnp.maximum(
        jnp.dot(ef_ref[...], we1t_ref[...], preferred_element_type=jnp.float32)
        + be1_ref[...], 0.0)
    ew = jnp.dot(g.astype(jnp.bfloat16), we2t_ref[...],
                 preferred_element_type=jnp.float32)
    hs = hs_ref[...]
    hrep = jnp.dot(hs.astype(jnp.bfloat16), rm_ref[...],
                   preferred_element_type=jnp.float32)
    p = hrep * ew
    w = H * H
    while w > H:
        w //= 2
        p = p[:, :w] + p[:, w:]
    msg_ref[...] = p + jnp.dot(hs, bb_ref[...],
                               preferred_element_type=jnp.float32)


@functools.cache
def _tc_msg():
    return pl.pallas_call(
        _msg_body,
        grid=(-(-N_EDGES // EB),),
        in_specs=[
            pl.BlockSpec((EB, D_EDGE), lambda i: (i, 0)),
            pl.BlockSpec((EB, H), lambda i: (i, 0)),
            pl.BlockSpec((D_EDGE, EH), lambda i: (0, 0)),
            pl.BlockSpec((1, EH), lambda i: (0, 0)),
            pl.BlockSpec((EH, H * H), lambda i: (0, 0)),
            pl.BlockSpec((H, H * H), lambda i: (0, 0)),
            pl.BlockSpec((H, H), lambda i: (0, 0)),
        ],
        out_specs=pl.BlockSpec((EB, H), lambda i: (i, 0)),
        out_shape=jax.ShapeDtypeStruct((EP, H), jnp.float32))


def _gru_body(parts_ref, h_ref, bc_ref, wih_ref, whh_ref, bih_ref,
              bhh_ref, o_ref):
    pa = parts_ref[0, :N_NODES, :]
    pb = parts_ref[1, :N_NODES, :]
    m = jnp.maximum(pa + pb + bc_ref[...], 0.0)
    gi = jnp.dot(m, wih_ref[...], preferred_element_type=jnp.float32) + bih_ref[...]
    h = h_ref[...]
    gh = jnp.dot(h, whh_ref[...], preferred_element_type=jnp.float32) + bhh_ref[...]
    r = jax.nn.sigmoid(gi[:, :H] + gh[:, :H])
    z = jax.nn.sigmoid(gi[:, H:2 * H] + gh[:, H:2 * H])
    ng = jnp.tanh(gi[:, 2 * H:] + r * gh[:, 2 * H:])
    o_ref[...] = (1.0 - z) * ng + z * h


@functools.cache
def _tc_gru():
    return pl.pallas_call(
        _gru_body,
        out_shape=jax.ShapeDtypeStruct((N_NODES, H), jnp.float32))


def _s2s_body(parts_ref, h_ref, bc_ref, wih_ref, whh_ref, bih_ref,
              bhh_ref, wih0_ref, whh0_ref, b0_ref, wih1_ref,
              whh1_ref, b1l_ref, wih2_ref, whh2_ref, b2l_ref, w1t_ref, b1_ref,
              w2t_ref, b2_ref, pred_ref, ro_ref):
    pa = parts_ref[0, :N_NODES, :]
    pb = parts_ref[1, :N_NODES, :]
    m = jnp.maximum(pa + pb + bc_ref[...], 0.0)
    gi = jnp.dot(m, wih_ref[...], preferred_element_type=jnp.float32) + bih_ref[...]
    h1 = h_ref[...]
    gh = jnp.dot(h1, whh_ref[...], preferred_element_type=jnp.float32) + bhh_ref[...]
    r = jax.nn.sigmoid(gi[:, :H] + gh[:, :H])
    z = jax.nn.sigmoid(gi[:, H:2 * H] + gh[:, H:2 * H])
    ng = jnp.tanh(gi[:, 2 * H:] + r * gh[:, 2 * H:])
    nodes = (1.0 - z) * ng + z * h1
    nodes_t = nodes.T
    wih = (wih0_ref[...], wih1_ref[...], wih2_ref[...])
    whh = (whh0_ref[...], whh1_ref[...], whh2_ref[...])
    bl = (b0_ref[...], b1l_ref[...], b2l_ref[...])
    q_star = jnp.zeros((1, 2 * H), jnp.float32)
    hs = [jnp.zeros((1, H), jnp.float32) for _ in range(3)]
    cs = [jnp.zeros((1, H), jnp.float32) for _ in range(3)]
    for _ in range(6):
        x = q_star
        for l in range(3):
            gates = (jnp.dot(x, wih[l], preferred_element_type=jnp.float32)
                     + jnp.dot(hs[l], whh[l], preferred_element_type=jnp.float32)
                     + bl[l])
            ig = jax.nn.sigmoid(gates[:, :H])
            fg = jax.nn.sigmoid(gates[:, H:2 * H])
            gg = jnp.tanh(gates[:, 2 * H:3 * H])
            og = jax.nn.sigmoid(gates[:, 3 * H:])
            cs[l] = fg * cs[l] + ig * gg
            hs[l] = og * jnp.tanh(cs[l])
            x = hs[l]
        q = x
        e = jnp.dot(q, nodes_t, preferred_element_type=jnp.float32)
        mx = jnp.max(e, axis=1, keepdims=True)
        ex = jnp.exp(e - mx)
        alpha = ex / jnp.sum(ex, axis=1, keepdims=True)
        rdt = jnp.dot(alpha, nodes, preferred_element_type=jnp.float32)
        q_star = jnp.concatenate([q, rdt], axis=1)
    ro_ref[...] = q_star
    hid = jnp.maximum(
        jnp.dot(q_star, w1t_ref[...], preferred_element_type=jnp.float32)
        + b1_ref[...], 0.0)
    pred_ref[...] = (jnp.dot(hid, w2t_ref[...], preferred_element_type=jnp.float32)
                     + b2_ref[...])


@functools.cache
def _tc_s2s():
    return pl.pallas_call(
        _s2s_body,
        out_shape=(jax.ShapeDtypeStruct((1, OUT), jnp.float32),
                   jax.ShapeDtypeStruct((1, 2 * H), jnp.float32)))


def kernel(n_feat, e_feat, edge_index, W0, b0, We1, be1, We2, be2, b_conv,
           gru_Wih, gru_Whh, gru_bih, gru_bhh,
           lstm_Wih0, lstm_Whh0, lstm_bih0, lstm_bhh0,
           lstm_Wih1, lstm_Whh1, lstm_bih1, lstm_bhh1,
           lstm_Wih2, lstm_Whh2, lstm_bih2, lstm_bhh2,
           W1, b1, W2, b2):
    pad = EP - N_EDGES
    src = jnp.concatenate([edge_index[0], jnp.zeros((pad,), jnp.int32)])
    dst = jnp.concatenate([edge_index[1], jnp.full((pad,), N_NODES, jnp.int32)])
    dst3 = dst.reshape(NW, NCH, CHUNK)
    zeros_acc = jnp.zeros((PZ, H), jnp.float32)

    h = _tc_init()(n_feat, W0.T, b0.reshape(1, H))

    we1t = We1.T
    be1r = be1.reshape(1, EH)
    we2t = We2.T.astype(jnp.bfloat16)
    rm = jnp.repeat(jnp.eye(H, dtype=jnp.bfloat16), H, axis=1)
    bb = be2.reshape(H, H)
    wih = gru_Wih.T
    whh = gru_Whh.T
    bih = gru_bih.reshape(1, 3 * H)
    bhh = gru_bhh.reshape(1, 3 * H)
    bc = b_conv.reshape(1, H)

    def mp(hcur):
        hs = _sc_gather()(hcur, src)
        msg = _tc_msg()(e_feat, hs, we1t, be1r, we2t, rm, bb)
        return _sc_scatter()(msg, dst3, zeros_acc)

    parts = mp(h)
    h = _tc_gru()(parts, h, bc, wih, whh, bih, bhh)
    parts = mp(h)

    pred, readout = _tc_s2s()(
        parts, h, bc, wih, whh, bih, bhh,
        lstm_Wih0.T, lstm_Whh0.T, (lstm_bih0 + lstm_bhh0).reshape(1, 4 * H),
        lstm_Wih1.T, lstm_Whh1.T, (lstm_bih1 + lstm_bhh1).reshape(1, 4 * H),
        lstm_Wih2.T, lstm_Whh2.T, (lstm_bih2 + lstm_bhh2).reshape(1, 4 * H),
        W1.T, b1.reshape(1, H), W2.T, b2.reshape(1, OUT))
    return (pred, readout)


# speedup vs baseline: 1.2533x; 1.2533x over previous
import functools

import jax
import jax.numpy as jnp
from jax import lax
from jax.experimental import pallas as pl
from jax.experimental.pallas import tpu as pltpu
from jax.experimental.pallas import tpu_sc as plsc

N_NODES = 10000
N_EDGES = 160000
D_NODE = 128
D_EDGE = 16
H = 32
EH = 64
OUT = 12

NC = 2
NS = 16
NW = NC * NS
CHUNK = 128
NCH = 40
HALF = NCH // 2
EPW = NCH * CHUNK
EP = NW * EPW
EB = 1024
PZ = 10016
RPS = PZ // NS


def _mesh():
    return plsc.VectorSubcoreMesh(
        core_axis_name="c", subcore_axis_name="s", num_cores=NC, num_subcores=NS)


@functools.cache
def _sc_gather():
    @functools.partial(
        pl.kernel,
        out_type=jax.ShapeDtypeStruct((EP, H), jnp.float32),
        mesh=_mesh(),
        compiler_params=pltpu.CompilerParams(use_tc_tiling_on_sc=False),
        scratch_types=[
            pltpu.VMEM((EPW,), jnp.int32),
            pltpu.VMEM((HALF * CHUNK, H), jnp.float32),
            pltpu.SemaphoreType.DMA,
        ],
    )
    def gather(tbl_hbm, idx_hbm, out_hbm, idx_v, rows_v, sem):
        c = lax.axis_index("c")
        s = lax.axis_index("s")
        w = s * NC + c
        pltpu.sync_copy(idx_hbm.at[pl.ds(w * EPW, EPW)], idx_v)
        for half in range(2):
            hc = HALF * CHUNK
            cp = pltpu.async_copy(
                tbl_hbm.at[idx_v.at[pl.ds(half * hc, hc)]], rows_v, sem)
            cp.wait()
            pltpu.sync_copy(
                rows_v, out_hbm.at[pl.ds(w * EPW + half * hc, hc)])

    return gather


@functools.cache
def _sc_scatter():
    @functools.partial(
        pl.kernel,
        out_type=jax.ShapeDtypeStruct((NC, PZ, H), jnp.float32),
        mesh=_mesh(),
        compiler_params=pltpu.CompilerParams(use_tc_tiling_on_sc=False),
        scratch_types=[
            pltpu.VMEM((NCH, CHUNK), jnp.int32),
            pltpu.VMEM((HALF * CHUNK, H), jnp.float32),
            pltpu.VMEM_SHARED((PZ, H), jnp.float32),
            pltpu.SemaphoreType.DMA,
        ],
    )
    def scatter(msg_hbm, idx_hbm, zeros_hbm, out_hbm, idx_v, msg_v, acc_sh, sem):
        c = lax.axis_index("c")
        s = lax.axis_index("s")
        w = s * NC + c
        pltpu.sync_copy(zeros_hbm.at[pl.ds(s * RPS, RPS)],
                        acc_sh.at[pl.ds(s * RPS, RPS)])
        pltpu.sync_copy(idx_hbm.at[w], idx_v)
        plsc.subcore_barrier()
        for half in range(2):
            hc = HALF * CHUNK
            cp = pltpu.async_copy(
                msg_hbm.at[pl.ds(w * EPW + half * hc, hc)], msg_v, sem)
            cp.wait()
            cps = []
            for b in range(HALF):
                cps.append(pltpu.async_copy(
                    msg_v.at[pl.ds(b * CHUNK, CHUNK)],
                    acc_sh.at[idx_v.at[half * HALF + b]], sem,
                    add=True))
            for cp2 in cps:
                cp2.wait()
        plsc.subcore_barrier()
        pltpu.sync_copy(acc_sh.at[pl.ds(s * RPS, RPS)],
                        out_hbm.at[c, pl.ds(s * RPS, RPS)])

    return scatter


def _init_body(x_ref, w_ref, b_ref, o_ref):
    o_ref[...] = jnp.maximum(
        jnp.dot(x_ref[...], w_ref[...], preferred_element_type=jnp.float32)
        + b_ref[...], 0.0)


@functools.cache
def _tc_init():
    return pl.pallas_call(
        _init_body,
        out_shape=jax.ShapeDtypeStruct((N_NODES, H), jnp.float32))


PK = 128 // H
EB4 = 512
LR_E = N_EDGES // PK
LR_P = EP // PK


def _msg_body(ef4_ref, hs4_ref, we1t_ref, be1_ref, we2t_ref, rm_ref, bb_ref,
              msg_ref):
    ef4 = ef4_ref[...]
    hs4 = hs4_ref[...]
    outs = []
    for j in range(PK):
        efj = ef4[:, j * D_EDGE:(j + 1) * D_EDGE]
        gj = jnp.maximum(
            jnp.dot(efj, we1t_ref[...], preferred_element_type=jnp.float32)
            + be1_ref[...], 0.0)
        ewj = jnp.dot(gj.astype(jnp.bfloat16), we2t_ref[...],
                      preferred_element_type=jnp.float32)
        hsj = hs4[:, j * H:(j + 1) * H]
        hrepj = jnp.dot(hsj.astype(jnp.bfloat16), rm_ref[...],
                        preferred_element_type=jnp.float32)
        p = hrepj * ewj
        w = H * H
        while w > H:
            w //= 2
            p = p[:, :w] + p[:, w:]
        outs.append(p + jnp.dot(hsj, bb_ref[...],
                                preferred_element_type=jnp.float32))
    msg_ref[...] = jnp.concatenate(outs, axis=1)


@functools.cache
def _tc_msg():
    return pl.pallas_call(
        _msg_body,
        grid=(-(-LR_E // EB4),),
        in_specs=[
            pl.BlockSpec((EB4, PK * D_EDGE), lambda i: (i, 0)),
            pl.BlockSpec((EB4, 128), lambda i: (i, 0)),
            pl.BlockSpec((D_EDGE, EH), lambda i: (0, 0)),
            pl.BlockSpec((1, EH), lambda i: (0, 0)),
            pl.BlockSpec((EH, H * H), lambda i: (0, 0)),
            pl.BlockSpec((H, H * H), lambda i: (0, 0)),
            pl.BlockSpec((H, H), lambda i: (0, 0)),
        ],
        out_specs=pl.BlockSpec((EB4, 128), lambda i: (i, 0)),
        out_shape=jax.ShapeDtypeStruct((LR_P, 128), jnp.float32))


def _gru_body(parts_ref, h_ref, bc_ref, wih_ref, whh_ref, bih_ref,
              bhh_ref, o_ref):
    pa = parts_ref[0, :N_NODES, :]
    pb = parts_ref[1, :N_NODES, :]
    m = jnp.maximum(pa + pb + bc_ref[...], 0.0)
    gi = jnp.dot(m, wih_ref[...], preferred_element_type=jnp.float32) + bih_ref[...]
    h = h_ref[...]
    gh = jnp.dot(h, whh_ref[...], preferred_element_type=jnp.float32) + bhh_ref[...]
    r = jax.nn.sigmoid(gi[:, :H] + gh[:, :H])
    z = jax.nn.sigmoid(gi[:, H:2 * H] + gh[:, H:2 * H])
    ng = jnp.tanh(gi[:, 2 * H:] + r * gh[:, 2 * H:])
    o_ref[...] = (1.0 - z) * ng + z * h


@functools.cache
def _tc_gru():
    return pl.pallas_call(
        _gru_body,
        out_shape=jax.ShapeDtypeStruct((N_NODES, H), jnp.float32))


def _s2s_body(parts_ref, h_ref, bc_ref, wih_ref, whh_ref, bih_ref,
              bhh_ref, wih0_ref, whh0_ref, b0_ref, wih1_ref,
              whh1_ref, b1l_ref, wih2_ref, whh2_ref, b2l_ref, w1t_ref, b1_ref,
              w2t_ref, b2_ref, pred_ref, ro_ref):
    pa = parts_ref[0, :N_NODES, :]
    pb = parts_ref[1, :N_NODES, :]
    m = jnp.maximum(pa + pb + bc_ref[...], 0.0)
    gi = jnp.dot(m, wih_ref[...], preferred_element_type=jnp.float32) + bih_ref[...]
    h1 = h_ref[...]
    gh = jnp.dot(h1, whh_ref[...], preferred_element_type=jnp.float32) + bhh_ref[...]
    r = jax.nn.sigmoid(gi[:, :H] + gh[:, :H])
    z = jax.nn.sigmoid(gi[:, H:2 * H] + gh[:, H:2 * H])
    ng = jnp.tanh(gi[:, 2 * H:] + r * gh[:, 2 * H:])
    nodes = (1.0 - z) * ng + z * h1
    nodes_t = nodes.T
    wih = (wih0_ref[...], wih1_ref[...], wih2_ref[...])
    whh = (whh0_ref[...], whh1_ref[...], whh2_ref[...])
    bl = (b0_ref[...], b1l_ref[...], b2l_ref[...])
    q_star = jnp.zeros((1, 2 * H), jnp.float32)
    hs = [jnp.zeros((1, H), jnp.float32) for _ in range(3)]
    cs = [jnp.zeros((1, H), jnp.float32) for _ in range(3)]
    for _ in range(6):
        x = q_star
        for l in range(3):
            gates = (jnp.dot(x, wih[l], preferred_element_type=jnp.float32)
                     + jnp.dot(hs[l], whh[l], preferred_element_type=jnp.float32)
                     + bl[l])
            ig = jax.nn.sigmoid(gates[:, :H])
            fg = jax.nn.sigmoid(gates[:, H:2 * H])
            gg = jnp.tanh(gates[:, 2 * H:3 * H])
            og = jax.nn.sigmoid(gates[:, 3 * H:])
            cs[l] = fg * cs[l] + ig * gg
            hs[l] = og * jnp.tanh(cs[l])
            x = hs[l]
        q = x
        e = jnp.dot(q, nodes_t, preferred_element_type=jnp.float32)
        mx = jnp.max(e, axis=1, keepdims=True)
        ex = jnp.exp(e - mx)
        alpha = ex / jnp.sum(ex, axis=1, keepdims=True)
        rdt = jnp.dot(alpha, nodes, preferred_element_type=jnp.float32)
        q_star = jnp.concatenate([q, rdt], axis=1)
    ro_ref[...] = q_star
    hid = jnp.maximum(
        jnp.dot(q_star, w1t_ref[...], preferred_element_type=jnp.float32)
        + b1_ref[...], 0.0)
    pred_ref[...] = (jnp.dot(hid, w2t_ref[...], preferred_element_type=jnp.float32)
                     + b2_ref[...])


@functools.cache
def _tc_s2s():
    return pl.pallas_call(
        _s2s_body,
        out_shape=(jax.ShapeDtypeStruct((1, OUT), jnp.float32),
                   jax.ShapeDtypeStruct((1, 2 * H), jnp.float32)))


def kernel(n_feat, e_feat, edge_index, W0, b0, We1, be1, We2, be2, b_conv,
           gru_Wih, gru_Whh, gru_bih, gru_bhh,
           lstm_Wih0, lstm_Whh0, lstm_bih0, lstm_bhh0,
           lstm_Wih1, lstm_Whh1, lstm_bih1, lstm_bhh1,
           lstm_Wih2, lstm_Whh2, lstm_bih2, lstm_bhh2,
           W1, b1, W2, b2):
    pad = EP - N_EDGES
    src = jnp.concatenate([edge_index[0], jnp.zeros((pad,), jnp.int32)])
    dst = jnp.concatenate([edge_index[1], jnp.full((pad,), N_NODES, jnp.int32)])
    dst3 = dst.reshape(NW, NCH, CHUNK)
    zeros_acc = jnp.zeros((PZ, H), jnp.float32)

    h = _tc_init()(n_feat, W0.T, b0.reshape(1, H))

    we1t = We1.T
    be1r = be1.reshape(1, EH)
    we2t = We2.T.astype(jnp.bfloat16)
    rm = jnp.repeat(jnp.eye(H, dtype=jnp.bfloat16), H, axis=1)
    bb = be2.reshape(H, H)
    wih = gru_Wih.T
    whh = gru_Whh.T
    bih = gru_bih.reshape(1, 3 * H)
    bhh = gru_bhh.reshape(1, 3 * H)
    bc = b_conv.reshape(1, H)

    ef4 = e_feat.reshape(LR_E, PK * D_EDGE)

    def mp(hcur):
        hs = _sc_gather()(hcur, src)
        msg128 = _tc_msg()(ef4, hs.reshape(LR_P, 128), we1t, be1r, we2t, rm, bb)
        return _sc_scatter()(msg128.reshape(EP, H), dst3, zeros_acc)

    parts = mp(h)
    h = _tc_gru()(parts, h, bc, wih, whh, bih, bhh)
    parts = mp(h)

    pred, readout = _tc_s2s()(
        parts, h, bc, wih, whh, bih, bhh,
        lstm_Wih0.T, lstm_Whh0.T, (lstm_bih0 + lstm_bhh0).reshape(1, 4 * H),
        lstm_Wih1.T, lstm_Whh1.T, (lstm_bih1 + lstm_bhh1).reshape(1, 4 * H),
        lstm_Wih2.T, lstm_Whh2.T, (lstm_bih2 + lstm_bhh2).reshape(1, 4 * H),
        W1.T, b1.reshape(1, H), W2.T, b2.reshape(1, OUT))
    return (pred, readout)


# speedup vs baseline: 1.3443x; 1.0726x over previous
import functools

import jax
import jax.numpy as jnp
from jax import lax
from jax.experimental import pallas as pl
from jax.experimental.pallas import tpu as pltpu
from jax.experimental.pallas import tpu_sc as plsc

N_NODES = 10000
N_EDGES = 160000
D_NODE = 128
D_EDGE = 16
H = 32
EH = 64
OUT = 12

NC = 2
NS = 16
NW = NC * NS
CHUNK = 128
NCH = 40
HALF = NCH // 2
EPW = NCH * CHUNK
EP = NW * EPW
EB = 1024
PZ = 10016
RPS = PZ // NS


def _mesh():
    return plsc.VectorSubcoreMesh(
        core_axis_name="c", subcore_axis_name="s", num_cores=NC, num_subcores=NS)


@functools.cache
def _sc_gather():
    @functools.partial(
        pl.kernel,
        out_type=jax.ShapeDtypeStruct((EP, H), jnp.float32),
        mesh=_mesh(),
        compiler_params=pltpu.CompilerParams(use_tc_tiling_on_sc=False),
        scratch_types=[
            pltpu.VMEM((EPW,), jnp.int32),
            pltpu.VMEM((HALF * CHUNK, H), jnp.float32),
            pltpu.SemaphoreType.DMA,
        ],
    )
    def gather(tbl_hbm, idx_hbm, out_hbm, idx_v, rows_v, sem):
        c = lax.axis_index("c")
        s = lax.axis_index("s")
        w = s * NC + c
        pltpu.sync_copy(idx_hbm.at[pl.ds(w * EPW, EPW)], idx_v)
        for half in range(2):
            hc = HALF * CHUNK
            cp = pltpu.async_copy(
                tbl_hbm.at[idx_v.at[pl.ds(half * hc, hc)]], rows_v, sem)
            cp.wait()
            pltpu.sync_copy(
                rows_v, out_hbm.at[pl.ds(w * EPW + half * hc, hc)])

    return gather


@functools.cache
def _sc_scatter():
    @functools.partial(
        pl.kernel,
        out_type=jax.ShapeDtypeStruct((NC, PZ, H), jnp.float32),
        mesh=_mesh(),
        compiler_params=pltpu.CompilerParams(use_tc_tiling_on_sc=False),
        scratch_types=[
            pltpu.VMEM((NCH, CHUNK), jnp.int32),
            pltpu.VMEM((HALF * CHUNK, H), jnp.float32),
            pltpu.VMEM_SHARED((PZ, H), jnp.float32),
            pltpu.SemaphoreType.DMA,
        ],
    )
    def scatter(msg_hbm, idx_hbm, zeros_hbm, out_hbm, idx_v, msg_v, acc_sh, sem):
        c = lax.axis_index("c")
        s = lax.axis_index("s")
        w = s * NC + c
        pltpu.sync_copy(zeros_hbm.at[pl.ds(s * RPS, RPS)],
                        acc_sh.at[pl.ds(s * RPS, RPS)])
        pltpu.sync_copy(idx_hbm.at[w], idx_v)
        plsc.subcore_barrier()
        for half in range(2):
            hc = HALF * CHUNK
            cp = pltpu.async_copy(
                msg_hbm.at[pl.ds(w * EPW + half * hc, hc)], msg_v, sem)
            cp.wait()
            cps = []
            for b in range(HALF):
                cps.append(pltpu.async_copy(
                    msg_v.at[pl.ds(b * CHUNK, CHUNK)],
                    acc_sh.at[idx_v.at[half * HALF + b]], sem,
                    add=True))
            for cp2 in cps:
                cp2.wait()
        plsc.subcore_barrier()
        pltpu.sync_copy(acc_sh.at[pl.ds(s * RPS, RPS)],
                        out_hbm.at[c, pl.ds(s * RPS, RPS)])

    return scatter


def _init_body(x_ref, w_ref, b_ref, o_ref):
    o_ref[...] = jnp.maximum(
        jnp.dot(x_ref[...], w_ref[...], preferred_element_type=jnp.float32)
        + b_ref[...], 0.0)


@functools.cache
def _tc_init():
    return pl.pallas_call(
        _init_body,
        out_shape=jax.ShapeDtypeStruct((N_NODES, H), jnp.float32))


PK = 128 // H
EB4 = 1024
LR_E = N_EDGES // PK
LR_P = EP // PK


def _msg_body(ef4_ref, hs4_ref, we1t_ref, be1_ref, we2t_ref, rm_ref, bb_ref,
              msg_ref):
    ef4 = ef4_ref[...]
    hs4 = hs4_ref[...]
    outs = []
    for j in range(PK):
        efj = ef4[:, j * D_EDGE:(j + 1) * D_EDGE]
        gj = jnp.maximum(
            jnp.dot(efj, we1t_ref[...], preferred_element_type=jnp.float32)
            + be1_ref[...], 0.0)
        ewj = jnp.dot(gj.astype(jnp.bfloat16), we2t_ref[...],
                      preferred_element_type=jnp.float32)
        hsj = hs4[:, j * H:(j + 1) * H]
        hrepj = jnp.dot(hsj.astype(jnp.bfloat16), rm_ref[...],
                        preferred_element_type=jnp.float32)
        p = hrepj * ewj
        w = H * H
        while w > H:
            w //= 2
            p = p[:, :w] + p[:, w:]
        outs.append(p + jnp.dot(hsj, bb_ref[...],
                                preferred_element_type=jnp.float32))
    msg_ref[...] = jnp.concatenate(outs, axis=1)


@functools.cache
def _tc_msg():
    return pl.pallas_call(
        _msg_body,
        grid=(-(-LR_E // EB4),),
        in_specs=[
            pl.BlockSpec((EB4, PK * D_EDGE), lambda i: (i, 0)),
            pl.BlockSpec((EB4, 128), lambda i: (i, 0)),
            pl.BlockSpec((D_EDGE, EH), lambda i: (0, 0)),
            pl.BlockSpec((1, EH), lambda i: (0, 0)),
            pl.BlockSpec((EH, H * H), lambda i: (0, 0)),
            pl.BlockSpec((H, H * H), lambda i: (0, 0)),
            pl.BlockSpec((H, H), lambda i: (0, 0)),
        ],
        out_specs=pl.BlockSpec((EB4, 128), lambda i: (i, 0)),
        out_shape=jax.ShapeDtypeStruct((LR_P, 128), jnp.float32))


def _gru_body(parts_ref, h_ref, bc_ref, wih_ref, whh_ref, bih_ref,
              bhh_ref, o_ref):
    pa = parts_ref[0, :N_NODES, :]
    pb = parts_ref[1, :N_NODES, :]
    m = jnp.maximum(pa + pb + bc_ref[...], 0.0)
    gi = jnp.dot(m, wih_ref[...], preferred_element_type=jnp.float32) + bih_ref[...]
    h = h_ref[...]
    gh = jnp.dot(h, whh_ref[...], preferred_element_type=jnp.float32) + bhh_ref[...]
    r = jax.nn.sigmoid(gi[:, :H] + gh[:, :H])
    z = jax.nn.sigmoid(gi[:, H:2 * H] + gh[:, H:2 * H])
    ng = jnp.tanh(gi[:, 2 * H:] + r * gh[:, 2 * H:])
    o_ref[...] = (1.0 - z) * ng + z * h


@functools.cache
def _tc_gru():
    return pl.pallas_call(
        _gru_body,
        out_shape=jax.ShapeDtypeStruct((N_NODES, H), jnp.float32))


def _s2s_body(parts_ref, h_ref, bc_ref, wih_ref, whh_ref, bih_ref,
              bhh_ref, wih0_ref, whh0_ref, b0_ref, wih1_ref,
              whh1_ref, b1l_ref, wih2_ref, whh2_ref, b2l_ref, w1t_ref, b1_ref,
              w2t_ref, b2_ref, pred_ref, ro_ref):
    pa = parts_ref[0, :N_NODES, :]
    pb = parts_ref[1, :N_NODES, :]
    m = jnp.maximum(pa + pb + bc_ref[...], 0.0)
    gi = jnp.dot(m, wih_ref[...], preferred_element_type=jnp.float32) + bih_ref[...]
    h1 = h_ref[...]
    gh = jnp.dot(h1, whh_ref[...], preferred_element_type=jnp.float32) + bhh_ref[...]
    r = jax.nn.sigmoid(gi[:, :H] + gh[:, :H])
    z = jax.nn.sigmoid(gi[:, H:2 * H] + gh[:, H:2 * H])
    ng = jnp.tanh(gi[:, 2 * H:] + r * gh[:, 2 * H:])
    nodes = (1.0 - z) * ng + z * h1
    nodes_t = nodes.T
    wih = (wih0_ref[...], wih1_ref[...], wih2_ref[...])
    whh = (whh0_ref[...], whh1_ref[...], whh2_ref[...])
    bl = (b0_ref[...], b1l_ref[...], b2l_ref[...])
    q_star = jnp.zeros((1, 2 * H), jnp.float32)
    hs = [jnp.zeros((1, H), jnp.float32) for _ in range(3)]
    cs = [jnp.zeros((1, H), jnp.float32) for _ in range(3)]
    for _ in range(6):
        x = q_star
        for l in range(3):
            gates = (jnp.dot(x, wih[l], preferred_element_type=jnp.float32)
                     + jnp.dot(hs[l], whh[l], preferred_element_type=jnp.float32)
                     + bl[l])
            ig = jax.nn.sigmoid(gates[:, :H])
            fg = jax.nn.sigmoid(gates[:, H:2 * H])
            gg = jnp.tanh(gates[:, 2 * H:3 * H])
            og = jax.nn.sigmoid(gates[:, 3 * H:])
            cs[l] = fg * cs[l] + ig * gg
            hs[l] = og * jnp.tanh(cs[l])
            x = hs[l]
        q = x
        e = jnp.dot(q, nodes_t, preferred_element_type=jnp.float32)
        mx = jnp.max(e, axis=1, keepdims=True)
        ex = jnp.exp(e - mx)
        alpha = ex / jnp.sum(ex, axis=1, keepdims=True)
        rdt = jnp.dot(alpha, nodes, preferred_element_type=jnp.float32)
        q_star = jnp.concatenate([q, rdt], axis=1)
    ro_ref[...] = q_star
    hid = jnp.maximum(
        jnp.dot(q_star, w1t_ref[...], preferred_element_type=jnp.float32)
        + b1_ref[...], 0.0)
    pred_ref[...] = (jnp.dot(hid, w2t_ref[...], preferred_element_type=jnp.float32)
                     + b2_ref[...])


@functools.cache
def _tc_s2s():
    return pl.pallas_call(
        _s2s_body,
        out_shape=(jax.ShapeDtypeStruct((1, OUT), jnp.float32),
                   jax.ShapeDtypeStruct((1, 2 * H), jnp.float32)))


def kernel(n_feat, e_feat, edge_index, W0, b0, We1, be1, We2, be2, b_conv,
           gru_Wih, gru_Whh, gru_bih, gru_bhh,
           lstm_Wih0, lstm_Whh0, lstm_bih0, lstm_bhh0,
           lstm_Wih1, lstm_Whh1, lstm_bih1, lstm_bhh1,
           lstm_Wih2, lstm_Whh2, lstm_bih2, lstm_bhh2,
           W1, b1, W2, b2):
    pad = EP - N_EDGES
    src = jnp.concatenate([edge_index[0], jnp.zeros((pad,), jnp.int32)])
    dst = jnp.concatenate([edge_index[1], jnp.full((pad,), N_NODES, jnp.int32)])
    dst3 = dst.reshape(NW, NCH, CHUNK)
    zeros_acc = jnp.zeros((PZ, H), jnp.float32)

    h = _tc_init()(n_feat, W0.T, b0.reshape(1, H))

    we1t = We1.T.astype(jnp.bfloat16)
    be1r = be1.reshape(1, EH)
    we2t = We2.T.astype(jnp.bfloat16)
    rm = jnp.repeat(jnp.eye(H, dtype=jnp.bfloat16), H, axis=1)
    bb = be2.reshape(H, H)
    wih = gru_Wih.T
    whh = gru_Whh.T
    bih = gru_bih.reshape(1, 3 * H)
    bhh = gru_bhh.reshape(1, 3 * H)
    bc = b_conv.reshape(1, H)

    ef4 = e_feat.astype(jnp.bfloat16).reshape(LR_E, PK * D_EDGE)

    def mp(hcur):
        hs = _sc_gather()(hcur, src)
        msg128 = _tc_msg()(ef4, hs.reshape(LR_P, 128), we1t, be1r, we2t, rm, bb)
        return _sc_scatter()(msg128.reshape(EP, H), dst3, zeros_acc)

    parts = mp(h)
    h = _tc_gru()(parts, h, bc, wih, whh, bih, bhh)
    parts = mp(h)

    pred, readout = _tc_s2s()(
        parts, h, bc, wih, whh, bih, bhh,
        lstm_Wih0.T, lstm_Whh0.T, (lstm_bih0 + lstm_bhh0).reshape(1, 4 * H),
        lstm_Wih1.T, lstm_Whh1.T, (lstm_bih1 + lstm_bhh1).reshape(1, 4 * H),
        lstm_Wih2.T, lstm_Whh2.T, (lstm_bih2 + lstm_bhh2).reshape(1, 4 * H),
        W1.T, b1.reshape(1, H), W2.T, b2.reshape(1, OUT))
    return (pred, readout)
